# baseline (speedup 1.0000x reference)
import functools

import jax
import jax.numpy as jnp
from jax import lax
from jax.experimental import pallas as pl
from jax.experimental.pallas import tpu as pltpu
from jax.experimental.pallas import tpu_sc as plsc

D = 768
NV = D // 16
PAD_IDX = 1
EPS = 1e-5

NC = 2
NS = 16
NW = NC * NS
CS = 32
NCH = 8
NB = 4
INV_D = 1.0 / D
MAGIC = 0x5F3759DF


def _fused(ids_r, table, pos_sl, type0, gamma, beta, n_rows, s_len):
    mesh = plsc.VectorSubcoreMesh(core_axis_name="c", subcore_axis_name="s")
    s_per_w = s_len // NW
    rows_per_srow = s_len // CS

    @functools.partial(
        pl.kernel,
        mesh=mesh,
        out_type=jax.ShapeDtypeStruct((n_rows, D), jnp.float32),
        scratch_types=[
            pltpu.VMEM((NB * NCH, CS), jnp.int32),
            pltpu.VMEM((CS, D), jnp.float32),
            pltpu.VMEM((CS, D), jnp.float32),
            pltpu.VMEM((CS, D), jnp.float32),
            pltpu.VMEM((CS, D), jnp.float32),
            pltpu.VMEM((1, D), jnp.float32),
            pltpu.VMEM((1, D), jnp.float32),
            pltpu.VMEM((1, D), jnp.float32),
            pltpu.SemaphoreType.DMA,
            pltpu.SemaphoreType.DMA,
            pltpu.SemaphoreType.DMA,
            pltpu.SemaphoreType.DMA,
            pltpu.SemaphoreType.DMA,
            pltpu.SemaphoreType.DMA,
        ],
    )
    def k(ids_hbm, tab_hbm, pos_hbm, t_hbm, ga_hbm, be_hbm, out_hbm,
          idx_v, pbuf0, pbuf1, wbuf0, wbuf1, tbuf, gbuf, bbuf,
          gs0, gs1, os0, os1, ps0, ps1):
        wid = lax.axis_index("s") * NC + lax.axis_index("c")
        s0 = wid * s_per_w

        for b in range(NB):
            pltpu.sync_copy(
                ids_hbm.at[pl.ds(b * rows_per_srow + wid * NCH, NCH)],
                idx_v.at[pl.ds(b * NCH, NCH)])
        pltpu.sync_copy(t_hbm, tbuf)
        pltpu.sync_copy(ga_hbm, gbuf)
        pltpu.sync_copy(be_hbm, bbuf)

        pbufs = (pbuf0, pbuf1)
        wbufs = (wbuf0, wbuf1)
        gsem = (gs0, gs1)
        osem = (os0, os1)
        psem = (ps0, ps1)

        pltpu.async_copy(pos_hbm.at[pl.ds(s0, CS)], pbuf0, ps0)
        pltpu.async_copy(pos_hbm.at[pl.ds(s0 + CS, CS)], pbuf1, ps1)
        pltpu.async_copy(tab_hbm.at[idx_v.at[0]], wbuf0, gs0)

        def compute_chunk(wb, pb, t, _):
            def p1(v, carry):
                s_, q_ = carry
                off = v * 16
                x = wb[t, pl.ds(off, 16)] + pb[t, pl.ds(off, 16)]
                wb[t, pl.ds(off, 16)] = x
                return (s_ + x, q_ + x * x)

            acc_s, acc_q = lax.fori_loop(
                0, NV, p1,
                (jnp.zeros((16,), jnp.float32), jnp.zeros((16,), jnp.float32)),
                unroll=8)
            def splat_sum(v16):
                r = v16
                for sh in (8, 4, 2, 1):
                    idx = (jnp.arange(16, dtype=jnp.int32) + sh) % 16
                    perm = lax.gather(
                        r, idx[:, None],
                        lax.GatherDimensionNumbers(
                            offset_dims=(), collapsed_slice_dims=(0,),
                            start_index_map=(0,)),
                        (1,), mode=lax.GatherScatterMode.PROMISE_IN_BOUNDS)
                    r = r + perm
                return r

            m_v = splat_sum(acc_s) * INV_D
            q_v = splat_sum(acc_q) * INV_D
            var_v = q_v - m_v * m_v + EPS
            iv = jnp.full((16,), MAGIC, jnp.int32) - lax.shift_right_logical(
                lax.bitcast_convert_type(var_v, jnp.int32),
                jnp.full((16,), 1, jnp.int32))
            y = lax.bitcast_convert_type(iv, jnp.float32)
            for _ in range(3):
                y = y * (1.5 - 0.5 * var_v * y * y)
            c1 = m_v * y

            def p2(v, carry):
                off = v * 16
                x = wb[t, pl.ds(off, 16)]
                g = gbuf[0, pl.ds(off, 16)]
                bb = bbuf[0, pl.ds(off, 16)]
                wb[t, pl.ds(off, 16)] = (x * y - c1) * g + bb
                return carry

            lax.fori_loop(0, NV, p2, 0, unroll=8)
            return 0

        def fold_type(pb, r, _):
            for v in range(NV):
                off = v * 16
                pb[r, pl.ds(off, 16)] = (
                    pb[r, pl.ds(off, 16)] + tbuf[0, pl.ds(off, 16)])
            return 0

        def c_body(c2, _):
            for cc in range(2):
                c = c2 * 2 + cc
                pb = pbufs[cc]
                pltpu.make_async_copy(
                    pos_hbm.at[pl.ds(s0 + c * CS, CS)], pb, psem[cc]).wait()
                lax.fori_loop(0, CS, functools.partial(fold_type, pb), 0)
                for b in range(NB):
                    i = c * NB + b
                    cur = b & 1
                    nxt = 1 - cur
                    r_idx = b * NCH + c
                    pltpu.make_async_copy(
                        tab_hbm.at[idx_v.at[r_idx]], wbufs[cur],
                        gsem[cur]).wait()
                    bp = b - 1 if b > 0 else NB - 1
                    cp = c if b > 0 else c - 1
                    rowp = bp * s_len + s0 + cp * CS

                    @pl.when(i >= 1)
                    def _():
                        pltpu.make_async_copy(
                            wbufs[nxt], out_hbm.at[pl.ds(rowp, CS)],
                            osem[nxt]).wait()

                    bn = b + 1 if b < NB - 1 else 0
                    cn = c if b < NB - 1 else c + 1
                    r_next = bn * NCH + cn

                    @pl.when(i + 1 < NCH * NB)
                    def _():
                        pltpu.async_copy(
                            tab_hbm.at[idx_v.at[r_next]], wbufs[nxt],
                            gsem[nxt])

                    lax.fori_loop(
                        0, CS, functools.partial(compute_chunk, wbufs[cur], pb),
                        0)
                    row = b * s_len + s0 + c * CS
                    pltpu.async_copy(
                        wbufs[cur], out_hbm.at[pl.ds(row, CS)], osem[cur])
                @pl.when(c + 2 < NCH)
                def _():
                    pltpu.async_copy(
                        pos_hbm.at[pl.ds(s0 + (c + 2) * CS, CS)], pb, psem[cc])
            return 0

        lax.fori_loop(0, NCH // 2, c_body, 0)

        pltpu.make_async_copy(
            wbuf1, out_hbm.at[pl.ds(3 * s_len + s0 + 7 * CS, CS)],
            osem[1]).wait()

    return k(ids_r, table, pos_sl, type0, gamma, beta)


def kernel(input_ids, word_embeddings, position_embeddings,
           token_type_embeddings, ln_gamma, ln_beta):
    b_sz, s_len = input_ids.shape
    ids_r = input_ids.astype(jnp.int32).reshape(-1, CS)
    pos_sl = position_embeddings[PAD_IDX + 1:PAD_IDX + 1 + s_len]
    out = _fused(
        ids_r, word_embeddings, pos_sl,
        token_type_embeddings[:1],
        ln_gamma.reshape(1, D), ln_beta.reshape(1, D),
        b_sz * s_len, s_len)
    return out.reshape(b_sz, s_len, D)


# speedup vs baseline: 3.5174x; 3.5174x over previous
import functools

import jax
import jax.numpy as jnp
from jax import lax
from jax.experimental import pallas as pl
from jax.experimental.pallas import tpu as pltpu
from jax.experimental.pallas import tpu_sc as plsc

D = 768
PAD_IDX = 1
EPS = 1e-5

NC = 2
NS = 16
NW = NC * NS
CHUNK = 64
NSLAB = 4


def _sc_gather(ids2d, table):
    n_chunks = ids2d.shape[0]
    ch_per_w = n_chunks // NW
    n_rows = n_chunks * CHUNK
    mesh = plsc.VectorSubcoreMesh(core_axis_name="c", subcore_axis_name="s")

    @functools.partial(
        pl.kernel,
        mesh=mesh,
        out_type=jax.ShapeDtypeStruct((n_rows, D), jnp.float32),
        scratch_types=[
            pltpu.VMEM((ch_per_w, CHUNK), jnp.int32),
            pltpu.VMEM((CHUNK, D), jnp.float32),
            pltpu.VMEM((CHUNK, D), jnp.float32),
            pltpu.SemaphoreType.DMA,
            pltpu.SemaphoreType.DMA,
            pltpu.SemaphoreType.DMA,
            pltpu.SemaphoreType.DMA,
        ],
    )
    def k(ids_hbm, tab_hbm, out_hbm, idx_v, buf0, buf1, g0, g1, o0, o1):
        wid = lax.axis_index("s") * NC + lax.axis_index("c")
        base = wid * ch_per_w
        pltpu.sync_copy(ids_hbm.at[pl.ds(base, ch_per_w)], idx_v)
        bufs = (buf0, buf1)
        gsem = (g0, g1)
        osem = (o0, o1)
        gcopy = [None, None]
        ocopy = [None, None]
        gcopy[0] = pltpu.async_copy(tab_hbm.at[idx_v.at[0]], buf0, g0)
        for c in range(ch_per_w):
            cur = c & 1
            nxt = 1 - cur
            gcopy[cur].wait()
            if c + 1 < ch_per_w:
                if ocopy[nxt] is not None:
                    ocopy[nxt].wait()
                gcopy[nxt] = pltpu.async_copy(
                    tab_hbm.at[idx_v.at[c + 1]], bufs[nxt], gsem[nxt])
            ocopy[cur] = pltpu.async_copy(
                bufs[cur], out_hbm.at[pl.ds((base + c) * CHUNK, CHUNK)],
                osem[cur])
        for b in range(2):
            if ocopy[b] is not None:
                ocopy[b].wait()

    return k(ids2d, table)


def _ln_body(g_ref, p_ref, t_ref, ga_ref, be_ref, *rest):
    o_ref = rest[-1]
    x = g_ref[...] + p_ref[...] + t_ref[...]
    mean = jnp.mean(x, axis=-1, keepdims=True)
    xc = x - mean
    var = jnp.mean(xc * xc, axis=-1, keepdims=True)
    o_ref[...] = xc * lax.rsqrt(var + EPS) * ga_ref[...] + be_ref[...]


def _tc_ln_slab(g_k, pos_k, type0, gamma, beta, out_prev, k, n_b, s_total):
    blk = 512
    sbk = pos_k.shape[0] // blk
    sb_total = s_total // blk
    n_rows = n_b * s_total

    base_specs = [
        pl.BlockSpec((blk, D), lambda s, b: (b * sbk + s, 0)),
        pl.BlockSpec((blk, D), lambda s, b: (s, 0)),
        pl.BlockSpec((1, D), lambda s, b: (0, 0)),
        pl.BlockSpec((1, D), lambda s, b: (0, 0)),
        pl.BlockSpec((1, D), lambda s, b: (0, 0)),
    ]
    out_spec = pl.BlockSpec(
        (blk, D), lambda s, b: (b * sb_total + k * sbk + s, 0))
    out_shape = jax.ShapeDtypeStruct((n_rows, D), jnp.float32)
    args = [g_k, pos_k, type0, gamma, beta]
    if out_prev is None:
        return pl.pallas_call(
            _ln_body, grid=(sbk, n_b), in_specs=base_specs,
            out_specs=out_spec, out_shape=out_shape,
        )(*args)
    return pl.pallas_call(
        _ln_body, grid=(sbk, n_b),
        in_specs=base_specs + [pl.BlockSpec(memory_space=pl.ANY)],
        out_specs=out_spec, out_shape=out_shape,
        input_output_aliases={5: 0},
    )(*args, out_prev)


def kernel(input_ids, word_embeddings, position_embeddings,
           token_type_embeddings, ln_gamma, ln_beta):
    b_sz, s_len = input_ids.shape
    slab_s = s_len // NSLAB
    ids32 = input_ids.astype(jnp.int32)
    pos_sl = position_embeddings[PAD_IDX + 1:PAD_IDX + 1 + s_len]
    type0 = token_type_embeddings[:1]
    gamma = ln_gamma.reshape(1, D)
    beta = ln_beta.reshape(1, D)
    gs = []
    for k in range(NSLAB):
        ids_k = ids32[:, k * slab_s:(k + 1) * slab_s].reshape(-1, CHUNK)
        gs.append(_sc_gather(ids_k, word_embeddings))
    out = None
    for k in range(NSLAB):
        out = _tc_ln_slab(
            gs[k], pos_sl[k * slab_s:(k + 1) * slab_s], type0, gamma, beta,
            out, k, b_sz, s_len)
    return out.reshape(b_sz, s_len, D)
